# Initial kernel scaffold; baseline (speedup 1.0000x reference)
import jax
import jax.numpy as jnp
from jax.experimental import pallas as pl


def kernel(subj_q, rel_q, obj_q, entity_embeddings, relation_embeddings, Wq_subj, Wq_rel, Wq_obj, Wk_e, Wv_e, Wk_r, Wv_r, logit_scale):
    raise NotImplementedError("write your pallas kernel here")


# speedup vs baseline: 2.9706x; 2.9706x over previous
import functools

import jax
import jax.numpy as jnp
from jax.experimental import pallas as pl
from jax.experimental.pallas import tpu as pltpu

D = 128
K = 10


def _prep_body(ls_ref, q_ref, wq_ref, wk_ref, wv_ref, qk_ref, qv_ref):
    q = q_ref[...]
    rn = jax.lax.rsqrt(jnp.sum(q * q, axis=1, keepdims=True))
    qn = q * rn
    qp = jax.lax.dot_general(qn, wq_ref[...], (((1,), (1,)), ((), ())),
                             preferred_element_type=jnp.float32)
    scale = jnp.exp(ls_ref[0])
    qk_ref[...] = scale * jax.lax.dot_general(qp, wk_ref[...], (((1,), (0,)), ((), ())),
                                              preferred_element_type=jnp.float32)
    qv_ref[...] = jax.lax.dot_general(qp, wv_ref[...], (((1,), (0,)), ((), ())),
                                      preferred_element_type=jnp.float32)


def _prep(q, wq, wk, wv, ls):
    B = q.shape[0]
    Bt = min(512, B)
    qk, qv = pl.pallas_call(
        _prep_body,
        grid=(B // Bt,),
        in_specs=[
            pl.BlockSpec(memory_space=pltpu.SMEM),
            pl.BlockSpec((Bt, D), lambda b: (b, 0)),
            pl.BlockSpec((D, D), lambda b: (0, 0)),
            pl.BlockSpec((D, D), lambda b: (0, 0)),
            pl.BlockSpec((D, D), lambda b: (0, 0)),
        ],
        out_specs=[pl.BlockSpec((Bt, D), lambda b: (b, 0))] * 2,
        out_shape=[jax.ShapeDtypeStruct((B, D), jnp.float32)] * 2,
    )(ls, q, wq, wk, wv)
    return qk, qv


def _retrieve_body(n_valid, nchunks, qk_ref, qv_ref, ent_ref, out_ref,
                   runv_ref, runw_ref):
    c = pl.program_id(1)
    C = ent_ref.shape[0]
    Bt = qk_ref.shape[0]

    @pl.when(c == 0)
    def _():
        runv_ref[...] = jnp.full_like(runv_ref, -jnp.inf)
        runw_ref[...] = jnp.zeros_like(runw_ref)

    ent = ent_ref[...]
    rn = jax.lax.rsqrt(jnp.sum(ent * ent, axis=1, keepdims=True))
    entn = ent * rn
    dims = (((1,), (1,)), ((), ()))
    lo = jax.lax.dot_general(qk_ref[...], entn, dims,
                             preferred_element_type=jnp.float32)
    wl = jax.lax.dot_general(qv_ref[...], entn, dims,
                             preferred_element_type=jnp.float32)
    col = jax.lax.broadcasted_iota(jnp.int32, (1, C), 1) + c * C
    lo = jnp.where(col < n_valid, lo, -jnp.inf)

    cand = jnp.concatenate([lo, runv_ref[...]], axis=1)
    wlc = jnp.concatenate([wl, runw_ref[...]], axis=1)
    lane = jax.lax.broadcasted_iota(jnp.int32, (Bt, 128), 1)
    newv = jnp.full((Bt, 128), -jnp.inf, jnp.float32)
    neww = jnp.zeros((Bt, 128), jnp.float32)
    for r in range(K):
        m = jnp.max(cand, axis=1, keepdims=True)
        eq = cand == m
        w = jnp.max(jnp.where(eq, wlc, -jnp.inf), axis=1, keepdims=True)
        cand = jnp.where(eq, -jnp.inf, cand)
        newv = jnp.where(lane == r, m, newv)
        neww = jnp.where(lane == r, w, neww)
    runv_ref[...] = newv
    runw_ref[...] = neww

    @pl.when(c == nchunks - 1)
    def _():
        mx = jnp.max(newv, axis=1, keepdims=True)
        e = jnp.exp(newv - mx)
        s = jnp.sum(e, axis=1, keepdims=True)
        score = jnp.sum((e / s) * neww, axis=1)
        out_ref[...] = score.reshape(out_ref.shape)


def _retrieve(ent, qk, qv, C):
    n = ent.shape[0]
    Btot = qk.shape[0]
    Bt = min(512, Btot)
    nchunks = pl.cdiv(n, C)
    out = pl.pallas_call(
        functools.partial(_retrieve_body, n, nchunks),
        grid=(Btot // Bt, nchunks),
        in_specs=[
            pl.BlockSpec((Bt, D), lambda b, c: (b, 0)),
            pl.BlockSpec((Bt, D), lambda b, c: (b, 0)),
            pl.BlockSpec((C, D), lambda b, c: (c, 0)),
        ],
        out_specs=pl.BlockSpec((1, Bt // 128, 128), lambda b, c: (b, 0, 0)),
        out_shape=jax.ShapeDtypeStruct((Btot // Bt, Bt // 128, 128), jnp.float32),
        scratch_shapes=[
            pltpu.VMEM((Bt, 128), jnp.float32),
            pltpu.VMEM((Bt, 128), jnp.float32),
        ],
    )(qk, qv, ent)
    return out.reshape(Btot)


def kernel(subj_q, rel_q, obj_q, entity_embeddings, relation_embeddings,
           Wq_subj, Wq_rel, Wq_obj, Wk_e, Wv_e, Wk_r, Wv_r, logit_scale):
    ls = jnp.reshape(logit_scale.astype(jnp.float32), (1,))
    qk_s, qv_s = _prep(subj_q, Wq_subj, Wk_e, Wv_e, ls)
    qk_o, qv_o = _prep(obj_q, Wq_obj, Wk_e, Wv_e, ls)
    qk_r, qv_r = _prep(rel_q, Wq_rel, Wk_r, Wv_r, ls)
    qk_so = jnp.concatenate([qk_s, qk_o], axis=0)
    qv_so = jnp.concatenate([qv_s, qv_o], axis=0)
    so = _retrieve(entity_embeddings, qk_so, qv_so, C=2048)
    r = _retrieve(relation_embeddings, qk_r, qv_r, C=1024)
    B = subj_q.shape[0]
    return jnp.stack([so[:B], r, so[B:]], axis=0)


# speedup vs baseline: 6.5070x; 2.1904x over previous
import functools

import jax
import jax.numpy as jnp
from jax.experimental import pallas as pl
from jax.experimental.pallas import tpu as pltpu

D = 128
K = 10


def _prep_body(ls_ref, q_ref, wq_ref, wk_ref, wv_ref, qk_ref, qv_ref):
    q = q_ref[...]
    rn = jax.lax.rsqrt(jnp.sum(q * q, axis=1, keepdims=True))
    qn = q * rn
    qp = jax.lax.dot_general(qn, wq_ref[...], (((1,), (1,)), ((), ())),
                             preferred_element_type=jnp.float32)
    scale = jnp.exp(ls_ref[0])
    qk_ref[...] = scale * jax.lax.dot_general(qp, wk_ref[...], (((1,), (0,)), ((), ())),
                                              preferred_element_type=jnp.float32)
    qv_ref[...] = jax.lax.dot_general(qp, wv_ref[...], (((1,), (0,)), ((), ())),
                                      preferred_element_type=jnp.float32)


def _prep(q, wq, wk, wv, ls):
    B = q.shape[0]
    Bt = min(512, B)
    qk, qv = pl.pallas_call(
        _prep_body,
        grid=(B // Bt,),
        in_specs=[
            pl.BlockSpec(memory_space=pltpu.SMEM),
            pl.BlockSpec((Bt, D), lambda b: (b, 0)),
            pl.BlockSpec((D, D), lambda b: (0, 0)),
            pl.BlockSpec((D, D), lambda b: (0, 0)),
            pl.BlockSpec((D, D), lambda b: (0, 0)),
        ],
        out_specs=[pl.BlockSpec((Bt, D), lambda b: (b, 0))] * 2,
        out_shape=[jax.ShapeDtypeStruct((B, D), jnp.float32)] * 2,
    )(ls, q, wq, wk, wv)
    return qk, qv


def _retrieve_body(n_valid, nchunks, qk_ref, qv_ref, ent_ref, out_ref,
                   r1_ref, r2_ref, r3_ref, w1_ref, w2_ref, w3_ref):
    c = pl.program_id(1)
    C = ent_ref.shape[0]
    Bt = qk_ref.shape[0]

    @pl.when(c == 0)
    def _():
        r1_ref[...] = jnp.full_like(r1_ref, -jnp.inf)
        r2_ref[...] = jnp.full_like(r2_ref, -jnp.inf)
        r3_ref[...] = jnp.full_like(r3_ref, -jnp.inf)
        w1_ref[...] = jnp.zeros_like(w1_ref)
        w2_ref[...] = jnp.zeros_like(w2_ref)
        w3_ref[...] = jnp.zeros_like(w3_ref)

    ent = ent_ref[...]
    rn = jax.lax.rsqrt(jnp.sum(ent * ent, axis=1, keepdims=True))
    entn = ent * rn
    dims = (((1,), (1,)), ((), ()))
    lo = jax.lax.dot_general(qk_ref[...], entn, dims,
                             preferred_element_type=jnp.float32)
    wl = jax.lax.dot_general(qv_ref[...], entn, dims,
                             preferred_element_type=jnp.float32)
    col = jax.lax.broadcasted_iota(jnp.int32, (1, C), 1) + c * C
    lo = jnp.where(col < n_valid, lo, -jnp.inf)

    r1 = r1_ref[...]
    r2 = r2_ref[...]
    r3 = r3_ref[...]
    w1 = w1_ref[...]
    w2 = w2_ref[...]
    w3 = w3_ref[...]
    for g in range(C // 128):
        x = lo[:, g * 128:(g + 1) * 128]
        wx = wl[:, g * 128:(g + 1) * 128]
        m = x > r1
        d = jnp.where(m, r1, x)
        dw = jnp.where(m, w1, wx)
        r1 = jnp.where(m, x, r1)
        w1 = jnp.where(m, wx, w1)
        m2 = d > r2
        d2 = jnp.where(m2, r2, d)
        dw2 = jnp.where(m2, w2, dw)
        r2 = jnp.where(m2, d, r2)
        w2 = jnp.where(m2, dw, w2)
        m3 = d2 > r3
        r3 = jnp.where(m3, d2, r3)
        w3 = jnp.where(m3, dw2, w3)
    r1_ref[...] = r1
    r2_ref[...] = r2
    r3_ref[...] = r3
    w1_ref[...] = w1
    w2_ref[...] = w2
    w3_ref[...] = w3

    @pl.when(c == nchunks - 1)
    def _():
        cand = jnp.concatenate([r1, r2, r3], axis=1)
        wlc = jnp.concatenate([w1, w2, w3], axis=1)
        lane = jax.lax.broadcasted_iota(jnp.int32, (Bt, 128), 1)
        newv = jnp.full((Bt, 128), -jnp.inf, jnp.float32)
        neww = jnp.zeros((Bt, 128), jnp.float32)
        for r in range(K):
            m = jnp.max(cand, axis=1, keepdims=True)
            eq = cand == m
            w = jnp.max(jnp.where(eq, wlc, -jnp.inf), axis=1, keepdims=True)
            cand = jnp.where(eq, -jnp.inf, cand)
            newv = jnp.where(lane == r, m, newv)
            neww = jnp.where(lane == r, w, neww)
        mx = jnp.max(newv, axis=1, keepdims=True)
        e = jnp.exp(newv - mx)
        s = jnp.sum(e, axis=1, keepdims=True)
        score = jnp.sum((e / s) * neww, axis=1)
        out_ref[...] = score.reshape(out_ref.shape)


def _retrieve(ent, qk, qv, C):
    n = ent.shape[0]
    Btot = qk.shape[0]
    Bt = min(512, Btot)
    nchunks = pl.cdiv(n, C)
    out = pl.pallas_call(
        functools.partial(_retrieve_body, n, nchunks),
        grid=(Btot // Bt, nchunks),
        in_specs=[
            pl.BlockSpec((Bt, D), lambda b, c: (b, 0)),
            pl.BlockSpec((Bt, D), lambda b, c: (b, 0)),
            pl.BlockSpec((C, D), lambda b, c: (c, 0)),
        ],
        out_specs=pl.BlockSpec((1, Bt // 128, 128), lambda b, c: (b, 0, 0)),
        out_shape=jax.ShapeDtypeStruct((Btot // Bt, Bt // 128, 128), jnp.float32),
        scratch_shapes=[pltpu.VMEM((Bt, 128), jnp.float32)] * 6,
    )(qk, qv, ent)
    return out.reshape(Btot)


def kernel(subj_q, rel_q, obj_q, entity_embeddings, relation_embeddings,
           Wq_subj, Wq_rel, Wq_obj, Wk_e, Wv_e, Wk_r, Wv_r, logit_scale):
    ls = jnp.reshape(logit_scale.astype(jnp.float32), (1,))
    qk_s, qv_s = _prep(subj_q, Wq_subj, Wk_e, Wv_e, ls)
    qk_o, qv_o = _prep(obj_q, Wq_obj, Wk_e, Wv_e, ls)
    qk_r, qv_r = _prep(rel_q, Wq_rel, Wk_r, Wv_r, ls)
    qk_so = jnp.concatenate([qk_s, qk_o], axis=0)
    qv_so = jnp.concatenate([qv_s, qv_o], axis=0)
    so = _retrieve(entity_embeddings, qk_so, qv_so, C=2048)
    r = _retrieve(relation_embeddings, qk_r, qv_r, C=1024)
    B = subj_q.shape[0]
    return jnp.stack([so[:B], r, so[B:]], axis=0)
